# baseline (speedup 1.0000x reference)
import functools

import jax
import jax.numpy as jnp
from jax import lax
from jax.experimental import pallas as pl
from jax.experimental.pallas import tpu as pltpu
from jax.experimental.pallas import tpu_sc as plsc

B, C, H, W = 4, 96, 384, 384
BC = B * C
HW = H * W
K = 4096
NC, NS = 2, 16
NW = NC * NS
PPW = BC // NW
LANES = 16


@jax.jit
def _sc_gather(xt, sidx, order):
    mesh = plsc.VectorSubcoreMesh(core_axis_name="c", subcore_axis_name="s")

    @functools.partial(
        pl.kernel,
        out_type=jax.ShapeDtypeStruct((BC, K), jnp.float32),
        compiler_params=pltpu.CompilerParams(use_tc_tiling_on_sc=False),
        mesh=mesh,
        scratch_types=[
            pltpu.VMEM((K,), jnp.int32),
            pltpu.VMEM((K,), jnp.int32),
            pltpu.VMEM((PPW, K), jnp.float32),
            pltpu.SemaphoreType.DMA,
        ],
    )
    def k(x_hbm, sidx_hbm, ord_hbm, out_hbm, sid_v, ord_v, sg_v, gsem):
        wid = lax.axis_index("s") * NC + lax.axis_index("c")
        pltpu.sync_copy(sidx_hbm, sid_v)
        pltpu.sync_copy(ord_hbm, ord_v)

        base = wid * PPW
        for p in range(PPW):
            pltpu.async_copy(x_hbm.at[base + p].at[sid_v], sg_v.at[p], gsem)
        for p in range(PPW):
            pltpu.make_async_copy(x_hbm.at[base + p].at[sid_v], sg_v.at[p],
                                  gsem).wait()

        for p in range(PPW):
            pltpu.async_copy(sg_v.at[p], out_hbm.at[base + p].at[ord_v], gsem)
        for p in range(PPW):
            pltpu.make_async_copy(sg_v.at[p], out_hbm.at[base + p].at[ord_v],
                                  gsem).wait()

    return k(xt, sidx, order)


def kernel(x, loc):
    xt = x.reshape(BC, HW)
    loc32 = loc.astype(jnp.int32)
    fidx = loc32[0] * W + loc32[1]
    iota = jnp.arange(K, dtype=jnp.int32)
    sidx, order = lax.sort_key_val(fidx, iota)
    out = _sc_gather(xt, sidx, order)
    return out.reshape(B, C, K)


# speedup vs baseline: 6.3040x; 6.3040x over previous
import functools

import jax
import jax.numpy as jnp
from jax import lax
from jax.experimental import pallas as pl
from jax.experimental.pallas import tpu as pltpu
from jax.experimental.pallas import tpu_sc as plsc

B, C, H, W = 4, 96, 384, 384
BC = B * C
HW = H * W
K = 4096
NC, NS = 2, 16
NW = NC * NS
PPW = BC // NW
LANES = 16
NCH = 4
CHUNK = HW // NCH


@jax.jit
def _sc_gather(xt, sidx, order, gbounds):
    mesh = plsc.VectorSubcoreMesh(core_axis_name="c", subcore_axis_name="s")

    @functools.partial(
        pl.kernel,
        out_type=jax.ShapeDtypeStruct((BC, K), jnp.float32),
        compiler_params=pltpu.CompilerParams(use_tc_tiling_on_sc=False,
                                             needs_layout_passes=False),
        mesh=mesh,
        scratch_types=[
            pltpu.VMEM((K,), jnp.int32),
            pltpu.VMEM((K,), jnp.int32),
            pltpu.VMEM((16,), jnp.int32),
            pltpu.VMEM((K,), jnp.float32),
            pltpu.VMEM((CHUNK,), jnp.float32),
            pltpu.VMEM((CHUNK,), jnp.float32),
            pltpu.SemaphoreType.DMA,
            pltpu.SemaphoreType.DMA,
        ],
    )
    def k(x_hbm, sidx_hbm, ord_hbm, gb_hbm, out_hbm,
          sid_v, ord_v, gb_v, og_v, buf0, buf1, sem0, sem1):
        wid = lax.axis_index("s") * NC + lax.axis_index("c")
        pltpu.sync_copy(sidx_hbm, sid_v)
        pltpu.sync_copy(ord_hbm, ord_v)
        pltpu.sync_copy(gb_hbm, gb_v)

        base = wid * PPW
        bufs = (buf0, buf1)
        sems = (sem0, sem1)

        def chunk_src(p, c):
            return x_hbm.at[base + p].at[pl.ds(c * CHUNK, CHUNK)]

        pltpu.async_copy(chunk_src(0, 0), bufs[0], sems[0])

        def plane_loop(p, _):
            for c in range(NCH):
                slot = c % 2
                pltpu.make_async_copy(chunk_src(p, c), bufs[slot],
                                      sems[slot]).wait()
                nslot = (c + 1) % 2
                if c + 1 < NCH:
                    pltpu.async_copy(chunk_src(p, c + 1), bufs[nslot],
                                     sems[nslot])
                else:
                    @pl.when(p + 1 < PPW)
                    def _():
                        pltpu.async_copy(chunk_src(p + 1, 0), bufs[nslot],
                                         sems[nslot])

                lo = c * CHUNK
                buf = bufs[slot]

                def gbody(g, _, lo=lo, buf=buf):
                    s = pl.multiple_of(g * LANES, LANES)
                    iv = sid_v[pl.ds(s, LANES)]
                    ov = ord_v[pl.ds(s, LANES)]
                    m = (iv >= lo) & (iv < lo + CHUNK)
                    rel = jnp.where(m, iv - lo, 0)
                    vals = plsc.load_gather(buf, [rel], mask=m)
                    plsc.store_scatter(og_v, [ov], vals, mask=m)
                    return ()

                gvec = gb_v[pl.ds(0, LANES)]
                lax.fori_loop(gvec[2 * c], gvec[2 * c + 1], gbody, ())

            pltpu.sync_copy(og_v, out_hbm.at[base + p])
            return ()

        lax.fori_loop(0, PPW, plane_loop, ())

    return k(xt, sidx, order, gbounds)


def kernel(x, loc):
    xt = x.reshape(BC, HW)
    loc32 = loc.astype(jnp.int32)
    fidx = loc32[0] * W + loc32[1]
    iota = jnp.arange(K, dtype=jnp.int32)
    sidx, order = lax.sort_key_val(fidx, iota)
    cuts = jnp.searchsorted(sidx, CHUNK * jnp.arange(NCH + 1, dtype=jnp.int32))
    cuts = cuts.astype(jnp.int32)
    glo = cuts[:-1] // LANES
    ghi = (cuts[1:] + LANES - 1) // LANES
    gbounds = jnp.pad(jnp.stack([glo, ghi], axis=1).reshape(8), (0, 8))
    out = _sc_gather(xt, sidx, order, gbounds)
    return out.reshape(B, C, K)


# speedup vs baseline: 6.8076x; 1.0799x over previous
import functools

import jax
import jax.numpy as jnp
from jax import lax
from jax.experimental import pallas as pl
from jax.experimental.pallas import tpu as pltpu
from jax.experimental.pallas import tpu_sc as plsc

B, C, H, W = 4, 96, 384, 384
BC = B * C
HW = H * W
K = 4096
NC, NS = 2, 16
NW = NC * NS
PPW = BC // NW
LANES = 16
DP = 5
IP = PPW - DP
NCH = 6
CHUNK = HW // NCH


@jax.jit
def _sc_gather(xt, fidx, sidx, order, gbounds):
    mesh = plsc.VectorSubcoreMesh(core_axis_name="c", subcore_axis_name="s")

    @functools.partial(
        pl.kernel,
        out_type=jax.ShapeDtypeStruct((BC, K), jnp.float32),
        compiler_params=pltpu.CompilerParams(use_tc_tiling_on_sc=False,
                                             needs_layout_passes=False),
        mesh=mesh,
        scratch_types=[
            pltpu.VMEM((K,), jnp.int32),
            pltpu.VMEM((K,), jnp.int32),
            pltpu.VMEM((K,), jnp.int32),
            pltpu.VMEM((16,), jnp.int32),
            pltpu.VMEM((K,), jnp.float32),
            pltpu.VMEM((CHUNK,), jnp.float32),
            pltpu.VMEM((CHUNK,), jnp.float32),
            pltpu.VMEM((IP, K), jnp.float32),
            pltpu.SemaphoreType.DMA,
            pltpu.SemaphoreType.DMA,
            pltpu.SemaphoreType.DMA,
        ],
    )
    def k(x_hbm, fidx_hbm, sidx_hbm, ord_hbm, gb_hbm, out_hbm,
          fid_v, sid_v, ord_v, gb_v, og_v, buf0, buf1, ig_v,
          sem0, sem1, isem):
        wid = lax.axis_index("s") * NC + lax.axis_index("c")
        pltpu.sync_copy(fidx_hbm, fid_v)
        pltpu.sync_copy(sidx_hbm, sid_v)
        pltpu.sync_copy(ord_hbm, ord_v)
        pltpu.sync_copy(gb_hbm, gb_v)

        base = wid * PPW
        bufs = (buf0, buf1)
        sems = (sem0, sem1)

        def chunk_src(p, c):
            return x_hbm.at[base + p].at[pl.ds(c * CHUNK, CHUNK)]

        pltpu.async_copy(chunk_src(0, 0), bufs[0], sems[0])
        for q in range(IP):
            pltpu.async_copy(x_hbm.at[base + DP + q].at[fid_v], ig_v.at[q],
                             isem)

        def plane_loop(p, _):
            for c in range(NCH):
                slot = c % 2
                pltpu.make_async_copy(chunk_src(p, c), bufs[slot],
                                      sems[slot]).wait()
                nslot = (c + 1) % 2
                if c + 1 < NCH:
                    pltpu.async_copy(chunk_src(p, c + 1), bufs[nslot],
                                     sems[nslot])
                else:
                    @pl.when(p + 1 < DP)
                    def _():
                        pltpu.async_copy(chunk_src(p + 1, 0), bufs[nslot],
                                         sems[nslot])

                lo = c * CHUNK
                buf = bufs[slot]

                def gbody(g, _, lo=lo, buf=buf):
                    s = pl.multiple_of(g * LANES, LANES)
                    iv = sid_v[pl.ds(s, LANES)]
                    ov = ord_v[pl.ds(s, LANES)]
                    m = (iv >= lo) & (iv < lo + CHUNK)
                    rel = jnp.where(m, iv - lo, 0)
                    vals = plsc.load_gather(buf, [rel], mask=m)
                    plsc.store_scatter(og_v, [ov], vals, mask=m)
                    return ()

                gvec = gb_v[pl.ds(0, LANES)]
                lax.fori_loop(gvec[2 * c], gvec[2 * c + 1], gbody, ())

            pltpu.sync_copy(og_v, out_hbm.at[base + p])
            return ()

        lax.fori_loop(0, DP, plane_loop, ())

        for q in range(IP):
            pltpu.make_async_copy(x_hbm.at[base + DP + q].at[fid_v],
                                  ig_v.at[q], isem).wait()
        pltpu.sync_copy(ig_v, out_hbm.at[pl.ds(base + DP, IP)])

    return k(xt, fidx, sidx, order, gbounds)


def kernel(x, loc):
    xt = x.reshape(BC, HW)
    loc32 = loc.astype(jnp.int32)
    fidx = loc32[0] * W + loc32[1]
    iota = jnp.arange(K, dtype=jnp.int32)
    sidx, order = lax.sort_key_val(fidx, iota)
    cuts = jnp.searchsorted(sidx, CHUNK * jnp.arange(NCH + 1, dtype=jnp.int32))
    cuts = cuts.astype(jnp.int32)
    glo = cuts[:-1] // LANES
    ghi = (cuts[1:] + LANES - 1) // LANES
    gbounds = jnp.pad(jnp.stack([glo, ghi], axis=1).reshape(2 * NCH),
                      (0, 16 - 2 * NCH))
    out = _sc_gather(xt, fidx, sidx, order, gbounds)
    return out.reshape(B, C, K)


# speedup vs baseline: 7.2081x; 1.0588x over previous
import functools

import jax
import jax.numpy as jnp
from jax import lax
from jax.experimental import pallas as pl
from jax.experimental.pallas import tpu as pltpu
from jax.experimental.pallas import tpu_sc as plsc

B, C, H, W = 4, 96, 384, 384
BC = B * C
HW = H * W
K = 4096
NC, NS = 2, 16
NW = NC * NS
PPW = BC // NW
CH = 128
NCHUNK = K // CH
LANES = 16


@jax.jit
def _sc_gather(xt, loc):
    mesh = plsc.VectorSubcoreMesh(core_axis_name="c", subcore_axis_name="s")

    @functools.partial(
        pl.kernel,
        out_type=jax.ShapeDtypeStruct((BC, K), jnp.float32),
        compiler_params=pltpu.CompilerParams(use_tc_tiling_on_sc=False),
        mesh=mesh,
        scratch_types=[
            pltpu.VMEM((2, K), jnp.int32),
            pltpu.VMEM((K,), jnp.int32),
            pltpu.VMEM((K,), jnp.float32),
            pltpu.SemaphoreType.DMA,
        ],
    )
    def k(x_hbm, loc_hbm, out_hbm, loc_v, idx_v, gat_v, gsem):
        wid = lax.axis_index("s") * NC + lax.axis_index("c")
        pltpu.sync_copy(loc_hbm, loc_v)

        def cbody(i, _):
            s = pl.multiple_of(i * LANES, LANES)
            v0 = loc_v[0, pl.ds(s, LANES)]
            v1 = loc_v[1, pl.ds(s, LANES)]
            idx_v[pl.ds(s, LANES)] = v0 * W + v1
            return ()

        lax.fori_loop(0, K // LANES, cbody, ())

        def pbody(p, _):
            plane = wid * PPW + p
            src_plane = x_hbm.at[plane]

            pltpu.async_copy(src_plane.at[idx_v], gat_v, gsem)
            pltpu.make_async_copy(src_plane.at[idx_v], gat_v, gsem).wait()
            pltpu.sync_copy(gat_v, out_hbm.at[plane])
            return ()

        lax.fori_loop(0, PPW, pbody, ())

    return k(xt, loc)


def kernel(x, loc):
    xt = x.reshape(BC, HW)
    out = _sc_gather(xt, loc.astype(jnp.int32))
    return out.reshape(B, C, K)


# speedup vs baseline: 7.4541x; 1.0341x over previous
import functools

import jax
import jax.numpy as jnp
from jax import lax
from jax.experimental import pallas as pl
from jax.experimental.pallas import tpu as pltpu
from jax.experimental.pallas import tpu_sc as plsc

B, C, H, W = 4, 96, 384, 384
BC = B * C
HW = H * W
K = 4096
NC, NS = 2, 16
NW = NC * NS
PPW = BC // NW
CH = 128
NCHUNK = K // CH
LANES = 16


@jax.jit
def _sc_gather(xt, loc):
    mesh = plsc.VectorSubcoreMesh(core_axis_name="c", subcore_axis_name="s")

    @functools.partial(
        pl.kernel,
        out_type=jax.ShapeDtypeStruct((BC, K), jnp.float32),
        compiler_params=pltpu.CompilerParams(use_tc_tiling_on_sc=False),
        mesh=mesh,
        scratch_types=[
            pltpu.VMEM((2, K), jnp.int32),
            pltpu.VMEM((K,), jnp.int32),
            pltpu.VMEM((PPW, K), jnp.float32),
            pltpu.SemaphoreType.DMA,
        ],
    )
    def k(x_hbm, loc_hbm, out_hbm, loc_v, idx_v, gat_v, gsem):
        wid = lax.axis_index("s") * NC + lax.axis_index("c")
        pltpu.sync_copy(loc_hbm, loc_v)

        def cbody(i, _):
            s = pl.multiple_of(i * LANES, LANES)
            v0 = loc_v[0, pl.ds(s, LANES)]
            v1 = loc_v[1, pl.ds(s, LANES)]
            idx_v[pl.ds(s, LANES)] = v0 * W + v1
            return ()

        lax.fori_loop(0, K // LANES, cbody, ())

        base = wid * PPW
        for p in range(PPW):
            pltpu.async_copy(x_hbm.at[base + p].at[idx_v], gat_v.at[p], gsem)
        for p in range(PPW):
            pltpu.make_async_copy(x_hbm.at[base + p].at[idx_v], gat_v.at[p],
                                  gsem).wait()
        pltpu.sync_copy(gat_v, out_hbm.at[pl.ds(base, PPW)])

    return k(xt, loc)


def kernel(x, loc):
    xt = x.reshape(BC, HW)
    out = _sc_gather(xt, loc.astype(jnp.int32))
    return out.reshape(B, C, K)
